# baseline (speedup 1.0000x reference)
import jax
import jax.numpy as jnp
from jax import lax
from jax.experimental import pallas as pl
from jax.experimental.pallas import tpu as pltpu
from jax.experimental.pallas import tpu_sc as plsc

B, H, W, C = 8, 112, 112, 96
E = H * W * C
F = (2 * H) * (2 * W) * C

NC, NS, L = 2, 16, 16

NCHUNK = 4
H2 = 2 * H
RPB = 2 * W * C
CROWS = H2 // NCHUNK
CS = F // NCHUNK
ACC_T = CS // NS
E_T = E // NS
ROW = 128
ROWS_BLK = 21
BLK = ROWS_BLK * ROW
NBLK = E_T // BLK
ZB = 2_688
B_PER_CORE = B // NC


def _unpool_body(pool_hbm, ind_hbm, out_hbm,
                 ind_raw0, val_raw0, ind_raw1, val_raw1,
                 idx_st0, val_st0, idx_st1, val_st1, zbuf,
                 acc_sh, sem_i0, sem_i1, sem_s0, sem_s1):
    c = lax.axis_index("c")
    s = lax.axis_index("s")

    zero16 = jnp.zeros((L,), jnp.float32)

    def _z(i):
        zbuf[pl.ds(i * L, L)] = zero16
    pl.loop(0, ZB // L)(_z)

    raws = ((ind_raw0, val_raw0, sem_i0), (ind_raw1, val_raw1, sem_i1))
    stages = ((idx_st0, val_st0, sem_s0), (idx_st1, val_st1, sem_s1))

    def chunk_task(b, kk):
        lo = kk * CS
        in_base = b * E + s * E_T
        h2lo = kk * CROWS

        def _zero_row(rr):
            def _zero_piece(z):
                pltpu.sync_copy(
                    zbuf, acc_sh.at[pl.ds(rr * RPB + z * ZB, ZB)])
            pl.loop(0, RPB // ZB)(_zero_piece)
        pl.loop(s, CROWS, step=NS)(_zero_row)
        plsc.subcore_barrier()

        def fire_input(blk, p):
            ir, vr, sem = raws[p]
            base = in_base + (blk % NBLK) * BLK
            pltpu.async_copy(ind_hbm.at[pl.ds(base, BLK)], ir, sem)
            pltpu.async_copy(pool_hbm.at[pl.ds(base, BLK)], vr, sem)

        def wait_input(p):
            ir, vr, sem = raws[p]
            pltpu.make_async_copy(ind_hbm.at[pl.ds(0, BLK)], ir, sem).wait()
            pltpu.make_async_copy(pool_hbm.at[pl.ds(0, BLK)], vr, sem).wait()

        def drain_scatter(p):
            _, vs, sem = stages[p]
            for r in range(ROWS_BLK):
                pltpu.make_async_copy(
                    vs.at[r], acc_sh.at[pl.ds(0, ROW)], sem).wait()

        def do_block(p):
            ir, vr, _ = raws[p]
            xs, vs, sem = stages[p]

            def _row(r):
                for v in range(ROW // L):
                    o = r * ROW + v * L
                    idx = ir[pl.ds(o, L)]
                    val = vr[pl.ds(o, L)]
                    loc = idx - lo
                    ok = (idx >= lo) & (loc < CS)
                    xs[r, pl.ds(v * L, L)] = jnp.where(
                        ok, loc, lax.shift_right_logical(idx, 2))
                    vs[r, pl.ds(v * L, L)] = jnp.where(ok, val, 0.0)
            pl.loop(0, ROWS_BLK)(_row)

            for r in range(ROWS_BLK):
                pltpu.async_copy(vs.at[r], acc_sh.at[xs.at[r]], sem,
                                 add=True)

        fire_input(0, 0)
        fire_input(1, 1)

        def _pair(i, first):
            wait_input(0)
            if not first:
                drain_scatter(0)
            do_block(0)
            fire_input(i + 2, 0)
            wait_input(1)
            if not first:
                drain_scatter(1)
            do_block(1)
            fire_input(i + 3, 1)
        _pair(0, True)
        pl.loop(2, NBLK, step=2)(lambda i: _pair(i, False))

        wait_input(0)
        wait_input(1)
        drain_scatter(0)
        drain_scatter(1)

        plsc.subcore_barrier()

        def _flush(rr):
            pltpu.sync_copy(
                acc_sh.at[pl.ds(rr * RPB, RPB)],
                out_hbm.at[b, h2lo + rr])
        pl.loop(s, CROWS, step=NS)(_flush)

    def _batch(b_local):
        def _chunk(kk):
            chunk_task(c * B_PER_CORE + b_local, kk)
        pl.loop(0, NCHUNK)(_chunk)
    pl.loop(0, B_PER_CORE)(_batch)


@jax.jit
def _unpool(pool_flat, ind_flat):
    mesh = plsc.VectorSubcoreMesh(
        core_axis_name="c", subcore_axis_name="s",
        num_cores=NC, num_subcores=NS)
    return pl.kernel(
        _unpool_body,
        out_type=jax.ShapeDtypeStruct((B, H2, RPB), jnp.float32),
        mesh=mesh,
        scratch_types=[
            pltpu.VMEM((BLK,), jnp.int32),
            pltpu.VMEM((BLK,), jnp.float32),
            pltpu.VMEM((BLK,), jnp.int32),
            pltpu.VMEM((BLK,), jnp.float32),
            pltpu.VMEM((ROWS_BLK, ROW), jnp.int32),
            pltpu.VMEM((ROWS_BLK, ROW), jnp.float32),
            pltpu.VMEM((ROWS_BLK, ROW), jnp.int32),
            pltpu.VMEM((ROWS_BLK, ROW), jnp.float32),
            pltpu.VMEM((ZB,), jnp.float32),
            pltpu.VMEM_SHARED((CS,), jnp.float32),
            pltpu.SemaphoreType.DMA,
            pltpu.SemaphoreType.DMA,
            pltpu.SemaphoreType.DMA,
            pltpu.SemaphoreType.DMA,
        ],
    )(pool_flat, ind_flat)


def kernel(pool, ind):
    pool_flat = pool.reshape(B * E)
    ind_flat = ind.astype(jnp.int32).reshape(B * E)
    out = _unpool(pool_flat, ind_flat)
    return out.reshape(B, 2 * H, 2 * W, C)


# speedup vs baseline: 1.7385x; 1.7385x over previous
import jax
import jax.numpy as jnp
from jax import lax
from jax.experimental import pallas as pl
from jax.experimental.pallas import tpu as pltpu
from jax.experimental.pallas import tpu_sc as plsc

B, H, W, C = 8, 112, 112, 96
E = H * W * C
F = (2 * H) * (2 * W) * C

NC, NS, L = 2, 16, 16

NCHUNK = 3
CS = F // NCHUNK
ACC_T = CS // NS
E_T = E // NS
ROW = 128
ROWS_BLK = 21
BLK = ROWS_BLK * ROW
NBLK = E_T // BLK
ZB = 3_136
IR_B = H * C
IR_T = IR_B // NS
RB = BLK // W
B_PER_CORE = B // NC


def _unpool_body(pool_hbm, ind_hbm, out_hbm,
                 ind_raw0, val_raw0, ind_raw1, val_raw1,
                 idx_st0, val_st0, idx_st1, val_st1, zbuf,
                 acc_sh, sem_i0, sem_i1, sem_s0, sem_s1):
    c = lax.axis_index("c")
    s = lax.axis_index("s")

    zero16 = jnp.zeros((L,), jnp.float32)

    def _z(i):
        zbuf[pl.ds(i * L, L)] = zero16
    pl.loop(0, ZB // L)(_z)

    raws = ((ind_raw0, val_raw0, sem_i0), (ind_raw1, val_raw1, sem_i1))
    stages = ((idx_st0, val_st0, sem_s0), (idx_st1, val_st1, sem_s1))

    def chunk_task(b, kk):
        lo = kk * CS
        in_row = b * IR_B + s * IR_T
        out_base = b * F + lo

        def _zero_slice(z):
            pltpu.sync_copy(
                zbuf, acc_sh.at[pl.ds(s * ACC_T + z * ZB, ZB)])
        pl.loop(0, ACC_T // ZB)(_zero_slice)
        plsc.subcore_barrier()

        def fire_input(blk, p):
            ir, vr, sem = raws[p]
            base = in_row + (blk % NBLK) * RB
            pltpu.async_copy(ind_hbm.at[pl.ds(base, RB), :], ir, sem)
            pltpu.async_copy(pool_hbm.at[pl.ds(base, RB), :], vr, sem)

        def wait_input(p):
            ir, vr, sem = raws[p]
            pltpu.make_async_copy(
                ind_hbm.at[pl.ds(0, RB), :], ir, sem).wait()
            pltpu.make_async_copy(
                pool_hbm.at[pl.ds(0, RB), :], vr, sem).wait()

        def drain_scatter(p):
            _, vs, sem = stages[p]
            for r in range(ROWS_BLK):
                pltpu.make_async_copy(
                    vs.at[r], acc_sh.at[pl.ds(0, ROW)], sem).wait()

        def do_block(p):
            ir, vr, _ = raws[p]
            xs, vs, sem = stages[p]

            def _row(r):
                for v in range(W // L):
                    idx = ir[r, pl.ds(v * L, L)]
                    val = vr[r, pl.ds(v * L, L)]
                    loc = idx - lo
                    ok = (idx >= lo) & (loc < CS)
                    p = r * W + v * L
                    pr = lax.shift_right_logical(p, 7)
                    pc = lax.bitwise_and(p, ROW - 1)
                    xs[pr, pl.ds(pc, L)] = jnp.where(
                        ok, loc, lax.shift_right_logical(idx, 2))
                    vs[pr, pl.ds(pc, L)] = jnp.where(ok, val, 0.0)
            pl.loop(0, RB)(_row)

            for r in range(ROWS_BLK):
                pltpu.async_copy(vs.at[r], acc_sh.at[xs.at[r]], sem,
                                 add=True)

        fire_input(0, 0)
        fire_input(1, 1)

        def _pair(i, first):
            wait_input(0)
            if not first:
                drain_scatter(0)
            do_block(0)
            fire_input(i + 2, 0)
            wait_input(1)
            if not first:
                drain_scatter(1)
            do_block(1)
            fire_input(i + 3, 1)
        _pair(0, True)
        pl.loop(2, NBLK, step=2)(lambda i: _pair(i, False))

        wait_input(0)
        wait_input(1)
        drain_scatter(0)
        drain_scatter(1)

        plsc.subcore_barrier()
        pltpu.sync_copy(
            acc_sh.at[pl.ds(s * ACC_T, ACC_T)],
            out_hbm.at[pl.ds(out_base + s * ACC_T, ACC_T)])

    def _batch(b_local):
        def _chunk(kk):
            chunk_task(c * B_PER_CORE + b_local, kk)
        pl.loop(0, NCHUNK)(_chunk)
    pl.loop(0, B_PER_CORE)(_batch)


@jax.jit
def _unpool(pool_flat, ind_flat):
    mesh = plsc.VectorSubcoreMesh(
        core_axis_name="c", subcore_axis_name="s",
        num_cores=NC, num_subcores=NS)
    return pl.kernel(
        _unpool_body,
        out_type=jax.ShapeDtypeStruct((B * F,), jnp.float32),
        mesh=mesh,
        scratch_types=[
            pltpu.VMEM((RB, W), jnp.int32),
            pltpu.VMEM((RB, W), jnp.float32),
            pltpu.VMEM((RB, W), jnp.int32),
            pltpu.VMEM((RB, W), jnp.float32),
            pltpu.VMEM((ROWS_BLK, ROW), jnp.int32),
            pltpu.VMEM((ROWS_BLK, ROW), jnp.float32),
            pltpu.VMEM((ROWS_BLK, ROW), jnp.int32),
            pltpu.VMEM((ROWS_BLK, ROW), jnp.float32),
            pltpu.VMEM((ZB,), jnp.float32),
            pltpu.VMEM_SHARED((CS,), jnp.float32),
            pltpu.SemaphoreType.DMA,
            pltpu.SemaphoreType.DMA,
            pltpu.SemaphoreType.DMA,
            pltpu.SemaphoreType.DMA,
        ],
    )(pool_flat, ind_flat)


def kernel(pool, ind):
    pool2 = pool.transpose(0, 1, 3, 2).reshape(B * H * C, W)
    ind2 = ind.astype(jnp.int32).transpose(0, 1, 3, 2).reshape(B * H * C, W)
    out = _unpool(pool2, ind2)
    return out.reshape(B, 2 * H, 2 * W, C)
